# baseline (speedup 1.0000x reference)
import math
import jax
import jax.numpy as jnp
from jax.experimental import pallas as pl

N = 10000
E = 320000
EDIM = 128
NDIM = 256
H = 8
DK = NDIM // H

ROW_BLK = 1000


def _qkv_body(x_ref, src_ref, dst_ref, wq_ref, bq_ref, wk_ref, wv_ref,
              q_ref, k_ref, v_ref):
    x = x_ref[...]
    q_ref[...] = jnp.dot(x, wq_ref[...], preferred_element_type=jnp.float32) \
        + bq_ref[...] + src_ref[...]
    k_ref[...] = jnp.dot(x, wk_ref[...], preferred_element_type=jnp.float32)
    v_ref[...] = jnp.dot(x, wv_ref[...], preferred_element_type=jnp.float32) \
        + dst_ref[...]


def _qkv(x, src_x, dst_x, Wq, bq, Wk, Wv):
    grid = (N // ROW_BLK,)
    row_spec = pl.BlockSpec((ROW_BLK, EDIM), lambda i: (i, 0))
    row_spec_n = pl.BlockSpec((ROW_BLK, NDIM), lambda i: (i, 0))
    w_spec = pl.BlockSpec((EDIM, NDIM), lambda i: (0, 0))
    b_spec = pl.BlockSpec((1, NDIM), lambda i: (0, 0))
    out_sds = jax.ShapeDtypeStruct((N, NDIM), jnp.float32)
    return pl.pallas_call(
        _qkv_body,
        grid=grid,
        in_specs=[row_spec, row_spec_n, row_spec_n, w_spec, b_spec, w_spec,
                  w_spec],
        out_specs=[row_spec_n, row_spec_n, row_spec_n],
        out_shape=[out_sds, out_sds, out_sds],
    )(x, src_x, dst_x, Wq, bq.reshape(1, NDIM), Wk, Wv)


def _ln(x, g, b, eps=1e-5):
    m = jnp.mean(x, axis=-1, keepdims=True)
    v = jnp.mean((x - m) ** 2, axis=-1, keepdims=True)
    return (x - m) / jnp.sqrt(v + eps) * g + b


def _post_body(wv_ref, z_ref, x_ref, wo_ref, bo_ref, ln1g_ref, ln1b_ref,
               w1_ref, b1_ref, w2_ref, b2_ref, ln2g_ref, ln2b_ref, out_ref):
    wv = wv_ref[...]
    z = z_ref[...]
    zb = jnp.repeat(z, DK, axis=1)
    o = wv / (zb + 1e-12)
    x = x_ref[...]
    h1 = x + jnp.dot(o, wo_ref[...], preferred_element_type=jnp.float32) \
        + bo_ref[...]
    h1 = _ln(h1, ln1g_ref[...], ln1b_ref[...])
    h2 = jnp.maximum(
        jnp.dot(h1, w1_ref[...], preferred_element_type=jnp.float32)
        + b1_ref[...], 0.0)
    h3 = h1 + jnp.dot(h2, w2_ref[...], preferred_element_type=jnp.float32) \
        + b2_ref[...]
    out_ref[...] = _ln(h3, ln2g_ref[...], ln2b_ref[...])


def _post(wv, z, x, Wo, bo, ln1_g, ln1_b, W1, b1, W2, b2, ln2_g, ln2_b):
    grid = (N // ROW_BLK,)
    full = lambda r, c: pl.BlockSpec((r, c), lambda i: (0, 0))
    return pl.pallas_call(
        _post_body,
        grid=grid,
        in_specs=[
            pl.BlockSpec((ROW_BLK, NDIM), lambda i: (i, 0)),
            pl.BlockSpec((ROW_BLK, H), lambda i: (i, 0)),
            pl.BlockSpec((ROW_BLK, EDIM), lambda i: (i, 0)),
            full(NDIM, EDIM), full(1, EDIM), full(1, EDIM), full(1, EDIM),
            full(EDIM, 4 * EDIM), full(1, 4 * EDIM),
            full(4 * EDIM, EDIM), full(1, EDIM),
            full(1, EDIM), full(1, EDIM),
        ],
        out_specs=pl.BlockSpec((ROW_BLK, EDIM), lambda i: (i, 0)),
        out_shape=jax.ShapeDtypeStruct((N, EDIM), jnp.float32),
    )(wv, z, x, Wo, bo.reshape(1, EDIM), ln1_g.reshape(1, EDIM),
      ln1_b.reshape(1, EDIM), W1, b1.reshape(1, 4 * EDIM), W2,
      b2.reshape(1, EDIM), ln2_g.reshape(1, EDIM), ln2_b.reshape(1, EDIM))


def kernel(x, src_x, dst_x, edge_index, Wq, bq, Wk, Wv, Wo, bo, ln1_g, ln1_b,
           W1, b1, W2, b2, ln2_g, ln2_b):
    q, k, v = _qkv(x, src_x, dst_x, Wq, bq, Wk, Wv)
    src = edge_index[0]
    dst = edge_index[1]
    qh = q.reshape(N, H, DK)
    kh = k.reshape(N, H, DK)
    vh = v.reshape(N, H, DK)
    score = jnp.sum(kh[src] * qh[dst], axis=-1) / math.sqrt(DK)
    score = jnp.exp(jnp.clip(score, -10.0, 10.0))
    wv = jax.ops.segment_sum(vh[src] * score[:, :, None], dst, num_segments=N)
    z = jax.ops.segment_sum(score, dst, num_segments=N)
    return _post(wv.reshape(N, NDIM), z, x, Wo, bo, ln1_g, ln1_b, W1, b1, W2,
                 b2, ln2_g, ln2_b)


# speedup vs baseline: 6.0453x; 6.0453x over previous
import dataclasses
import functools
import math
import jax
import jax.numpy as jnp
from jax import lax
from jax.experimental import pallas as pl
from jax.experimental.pallas import tpu as pltpu
from jax.experimental.pallas import tpu_sc as plsc

N = 10000
E = 320000
EDIM = 128
NDIM = 256
H = 8
DK = NDIM // H

ROW_BLK = 1000
HALF = 128
CH = 80
GRP = 16
NTILES = 16
EPT = E // NTILES
N_PAD = 10112
ROWS_PT = N_PAD // NTILES
ZROWS = 512


def _qkv_body(x_ref, src_ref, dst_ref, wq_ref, bq_ref, wk_ref, wv_ref,
              q_ref, k_ref, v_ref):
    x = x_ref[...]
    q = jnp.dot(x, wq_ref[...], preferred_element_type=jnp.float32) \
        + bq_ref[...] + src_ref[...]
    k = jnp.dot(x, wk_ref[...], preferred_element_type=jnp.float32)
    v = jnp.dot(x, wv_ref[...], preferred_element_type=jnp.float32) \
        + dst_ref[...]
    q_ref[0], q_ref[1] = q[:, :HALF], q[:, HALF:]
    k_ref[0], k_ref[1] = k[:, :HALF], k[:, HALF:]
    v_ref[0], v_ref[1] = v[:, :HALF], v[:, HALF:]


def _qkv(x, src_x, dst_x, Wq, bq, Wk, Wv):
    grid = (N // ROW_BLK,)
    row_spec = pl.BlockSpec((ROW_BLK, EDIM), lambda i: (i, 0))
    row_spec_n = pl.BlockSpec((ROW_BLK, NDIM), lambda i: (i, 0))
    w_spec = pl.BlockSpec((EDIM, NDIM), lambda i: (0, 0))
    b_spec = pl.BlockSpec((1, NDIM), lambda i: (0, 0))
    out_spec = pl.BlockSpec((2, ROW_BLK, HALF), lambda i: (0, i, 0))
    out_sds = jax.ShapeDtypeStruct((2, N, HALF), jnp.float32)
    return pl.pallas_call(
        _qkv_body,
        grid=grid,
        in_specs=[row_spec, row_spec_n, row_spec_n, w_spec, b_spec, w_spec,
                  w_spec],
        out_specs=[out_spec, out_spec, out_spec],
        out_shape=[out_sds, out_sds, out_sds],
    )(x, src_x, dst_x, Wq, bq.reshape(1, NDIM), Wk, Wv)


_INV_SQRT_DK = 1.0 / math.sqrt(DK)


def _edge_sc_body(kf_hbm, qf_hbm, vf_hbm, srcp_hbm, dstp_hbm,
                  outwv_hbm, outz_hbm,
                  src_v, dq_v, dr_v, zr_v, kv_rows, q_rows,
                  payload, z_pay, acc, accz, sem):
    cid = lax.axis_index("c")
    sid = lax.axis_index("s")
    zeros16 = jnp.zeros((GRP,), jnp.float32)
    iota16 = lax.iota(jnp.int32, GRP)

    @pl.loop(0, CH)
    def _zrow(r):
        for j in range(HALF // GRP):
            z_pay[r, pl.ds(j * GRP, GRP)] = zeros16

    row0 = sid * ROWS_PT
    for j in range(ROWS_PT // CH):
        pltpu.sync_copy(z_pay, acc.at[pl.ds(row0 + j * CH, CH)])
    rem = ROWS_PT % CH
    if rem:
        pltpu.sync_copy(z_pay.at[pl.ds(0, rem)],
                        acc.at[pl.ds(row0 + ROWS_PT - rem, rem)])
    zrow0 = sid * (ZROWS // NTILES)
    pltpu.sync_copy(z_pay.at[pl.ds(0, ZROWS // NTILES)],
                    accz.at[pl.ds(zrow0, ZROWS // NTILES)])

    plsc.subcore_barrier()

    ebase = sid * EPT

    @pl.loop(0, EPT, step=CH)
    def _chunk(e0):
        base = ebase + e0
        pltpu.sync_copy(srcp_hbm.at[pl.ds(cid * E + base, CH)], src_v)
        pltpu.sync_copy(dstp_hbm.at[pl.ds(cid * E + base, CH)], dq_v)
        pltpu.sync_copy(dstp_hbm.at[pl.ds(base, CH)], dr_v)
        ck = pltpu.async_copy(kf_hbm.at[src_v], kv_rows, sem)
        cq = pltpu.async_copy(qf_hbm.at[dq_v], q_rows, sem)
        ck.wait()
        cq.wait()

        @pl.loop(0, CH, step=GRP)
        def _grp(g):
            eids = g + iota16
            dvec = dr_v[pl.ds(g, GRP)]
            zr_v[pl.ds(g, GRP)] = lax.shift_right_logical(dvec, 5)
            zcol0 = jnp.bitwise_and(dvec, 31) * 4
            for h in range(H // 2):
                acc_s = zeros16
                for dd in range(DK):
                    col = jnp.full((GRP,), h * DK + dd, jnp.int32)
                    kv = plsc.load_gather(kv_rows, [eids, col])
                    qv = plsc.load_gather(q_rows, [eids, col])
                    acc_s = acc_s + kv * qv
                s = jnp.exp(jnp.clip(acc_s * _INV_SQRT_DK, -10.0, 10.0))
                plsc.store_scatter(z_pay, [eids, zcol0 + h], s)

        cv = pltpu.async_copy(vf_hbm.at[src_v], kv_rows, sem)
        cv.wait()

        @pl.loop(0, CH, step=GRP)
        def _grp2(g):
            eids = g + iota16
            zcol0 = jnp.bitwise_and(dr_v[pl.ds(g, GRP)], 31) * 4
            for h in range(H // 2):
                s = plsc.load_gather(z_pay, [eids, zcol0 + h])
                for dd in range(DK):
                    col = jnp.full((GRP,), h * DK + dd, jnp.int32)
                    vv = plsc.load_gather(kv_rows, [eids, col])
                    plsc.store_scatter(payload, [eids, col], vv * s)

        pltpu.sync_copy(payload, acc.at[dr_v], add=True)
        pltpu.sync_copy(z_pay, accz.at[zr_v], add=True)

        @pl.loop(0, CH, step=GRP)
        def _zclr(g):
            eids = g + iota16
            zcol0 = jnp.bitwise_and(dr_v[pl.ds(g, GRP)], 31) * 4
            for h in range(H // 2):
                plsc.store_scatter(z_pay, [eids, zcol0 + h], zeros16)

    plsc.subcore_barrier()

    for j in range(ROWS_PT // CH):
        r = row0 + j * CH
        pltpu.sync_copy(acc.at[pl.ds(r, CH)], payload)
        pltpu.sync_copy(payload, outwv_hbm.at[pl.ds(cid * N_PAD + r, CH)])
    if ROWS_PT % CH:
        rem = ROWS_PT % CH
        r = row0 + ROWS_PT - rem
        pltpu.sync_copy(acc.at[pl.ds(r, rem)], payload.at[pl.ds(0, rem)])
        pltpu.sync_copy(payload.at[pl.ds(0, rem)],
                        outwv_hbm.at[pl.ds(cid * N_PAD + r, rem)])
    pltpu.sync_copy(accz.at[pl.ds(zrow0, ZROWS // NTILES)],
                    payload.at[pl.ds(0, ZROWS // NTILES)])
    pltpu.sync_copy(payload.at[pl.ds(0, ZROWS // NTILES)],
                    outz_hbm.at[pl.ds(cid * ZROWS + zrow0, ZROWS // NTILES)])


def _edge_sc(kf, qf, vf, srcp, dstp):
    mesh = plsc.VectorSubcoreMesh(core_axis_name="c", subcore_axis_name="s")
    cp = pltpu.CompilerParams()
    if "needs_layout_passes" in pltpu.CompilerParams.__dataclass_fields__:
        cp = dataclasses.replace(cp, needs_layout_passes=False)
    kern = functools.partial(
        pl.kernel,
        mesh=mesh,
        compiler_params=cp,
        out_type=[jax.ShapeDtypeStruct((2 * N_PAD, HALF), jnp.float32),
                  jax.ShapeDtypeStruct((2 * ZROWS, HALF), jnp.float32)],
        scratch_types=[
            pltpu.VMEM((CH,), jnp.int32),
            pltpu.VMEM((CH,), jnp.int32),
            pltpu.VMEM((CH,), jnp.int32),
            pltpu.VMEM((CH,), jnp.int32),
            pltpu.VMEM((CH, HALF), jnp.float32),
            pltpu.VMEM((CH, HALF), jnp.float32),
            pltpu.VMEM((CH, HALF), jnp.float32),
            pltpu.VMEM((CH, HALF), jnp.float32),
            pltpu.VMEM_SHARED((N_PAD, HALF), jnp.float32),
            pltpu.VMEM_SHARED((ZROWS, HALF), jnp.float32),
            pltpu.SemaphoreType.DMA,
        ],
    )(_edge_sc_body)
    return kern(kf, qf, vf, srcp, dstp)


def _ln(x, g, b, eps=1e-5):
    m = jnp.mean(x, axis=-1, keepdims=True)
    v = jnp.mean((x - m) ** 2, axis=-1, keepdims=True)
    return (x - m) / jnp.sqrt(v + eps) * g + b


def _post_body(wv_ref, z_ref, x_ref, wo_ref, bo_ref, ln1g_ref, ln1b_ref,
               w1_ref, b1_ref, w2_ref, b2_ref, ln2g_ref, ln2b_ref, out_ref):
    wv = wv_ref[...]
    z = z_ref[...]
    zb = jnp.repeat(z, DK, axis=1)
    o = wv / (zb + 1e-12)
    x = x_ref[...]
    h1 = x + jnp.dot(o, wo_ref[...], preferred_element_type=jnp.float32) \
        + bo_ref[...]
    h1 = _ln(h1, ln1g_ref[...], ln1b_ref[...])
    h2 = jnp.maximum(
        jnp.dot(h1, w1_ref[...], preferred_element_type=jnp.float32)
        + b1_ref[...], 0.0)
    h3 = h1 + jnp.dot(h2, w2_ref[...], preferred_element_type=jnp.float32) \
        + b2_ref[...]
    out_ref[...] = _ln(h3, ln2g_ref[...], ln2b_ref[...])


def _post(wv, z, x, Wo, bo, ln1_g, ln1_b, W1, b1, W2, b2, ln2_g, ln2_b):
    grid = (N // ROW_BLK,)
    full = lambda r, c: pl.BlockSpec((r, c), lambda i: (0, 0))
    return pl.pallas_call(
        _post_body,
        grid=grid,
        in_specs=[
            pl.BlockSpec((ROW_BLK, NDIM), lambda i: (i, 0)),
            pl.BlockSpec((ROW_BLK, H), lambda i: (i, 0)),
            pl.BlockSpec((ROW_BLK, EDIM), lambda i: (i, 0)),
            full(NDIM, EDIM), full(1, EDIM), full(1, EDIM), full(1, EDIM),
            full(EDIM, 4 * EDIM), full(1, 4 * EDIM),
            full(4 * EDIM, EDIM), full(1, EDIM),
            full(1, EDIM), full(1, EDIM),
        ],
        out_specs=pl.BlockSpec((ROW_BLK, EDIM), lambda i: (i, 0)),
        out_shape=jax.ShapeDtypeStruct((N, EDIM), jnp.float32),
    )(wv, z, x, Wo, bo.reshape(1, EDIM), ln1_g.reshape(1, EDIM),
      ln1_b.reshape(1, EDIM), W1, b1.reshape(1, 4 * EDIM), W2,
      b2.reshape(1, EDIM), ln2_g.reshape(1, EDIM), ln2_b.reshape(1, EDIM))


def kernel(x, src_x, dst_x, edge_index, Wq, bq, Wk, Wv, Wo, bo, ln1_g, ln1_b,
           W1, b1, W2, b2, ln2_g, ln2_b):
    q3, k3, v3 = _qkv(x, src_x, dst_x, Wq, bq, Wk, Wv)
    qf = q3.reshape(2 * N, HALF)
    kf = k3.reshape(2 * N, HALF)
    vf = v3.reshape(2 * N, HALF)
    src = edge_index[0]
    dst = edge_index[1]
    off = jnp.array([[0], [N]], jnp.int32)
    srcp = (src[None, :] + off).reshape(2 * E)
    dstp = (dst[None, :] + off).reshape(2 * E)
    acc_wv, acc_z = _edge_sc(kf, qf, vf, srcp, dstp)
    wv = jnp.concatenate([acc_wv[:N], acc_wv[N_PAD:N_PAD + N]], axis=1)
    z0 = acc_z[:ZROWS].reshape(ZROWS * 32, 4)[:N]
    z1 = acc_z[ZROWS:].reshape(ZROWS * 32, 4)[:N]
    z = jnp.concatenate([z0, z1], axis=1)
    return _post(wv, z, x, Wo, bo, ln1_g, ln1_b, W1, b1, W2, b2,
                 ln2_g, ln2_b)


# speedup vs baseline: 6.4630x; 1.0691x over previous
import dataclasses
import functools
import math
import jax
import jax.numpy as jnp
from jax import lax
from jax.experimental import pallas as pl
from jax.experimental.pallas import tpu as pltpu
from jax.experimental.pallas import tpu_sc as plsc

N = 10000
E = 320000
EDIM = 128
NDIM = 256
H = 8
DK = NDIM // H

ROW_BLK = 1000
HALF = 128
CH = 32
GRP = 16
NTILES = 16
EPT = E // NTILES
N_PAD = 10112
ROWS_PT = N_PAD // NTILES
ZROWS = 512


def _qkv_body(x_ref, src_ref, dst_ref, wq_ref, bq_ref, wk_ref, wv_ref,
              q_ref, k_ref, v_ref):
    x = x_ref[...]
    q = jnp.dot(x, wq_ref[...], preferred_element_type=jnp.float32) \
        + bq_ref[...] + src_ref[...]
    k = jnp.dot(x, wk_ref[...], preferred_element_type=jnp.float32)
    v = jnp.dot(x, wv_ref[...], preferred_element_type=jnp.float32) \
        + dst_ref[...]
    q_ref[0], q_ref[1] = q[:, :HALF], q[:, HALF:]
    k_ref[0], k_ref[1] = k[:, :HALF], k[:, HALF:]
    v_ref[0], v_ref[1] = v[:, :HALF], v[:, HALF:]


def _qkv(x, src_x, dst_x, Wq, bq, Wk, Wv):
    grid = (N // ROW_BLK,)
    row_spec = pl.BlockSpec((ROW_BLK, EDIM), lambda i: (i, 0))
    row_spec_n = pl.BlockSpec((ROW_BLK, NDIM), lambda i: (i, 0))
    w_spec = pl.BlockSpec((EDIM, NDIM), lambda i: (0, 0))
    b_spec = pl.BlockSpec((1, NDIM), lambda i: (0, 0))
    out_spec = pl.BlockSpec((2, ROW_BLK, HALF), lambda i: (0, i, 0))
    out_sds = jax.ShapeDtypeStruct((2, N, HALF), jnp.float32)
    return pl.pallas_call(
        _qkv_body,
        grid=grid,
        in_specs=[row_spec, row_spec_n, row_spec_n, w_spec, b_spec, w_spec,
                  w_spec],
        out_specs=[out_spec, out_spec, out_spec],
        out_shape=[out_sds, out_sds, out_sds],
    )(x, src_x, dst_x, Wq, bq.reshape(1, NDIM), Wk, Wv)


_INV_SQRT_DK = 1.0 / math.sqrt(DK)


def _edge_sc_body(kf_hbm, qf_hbm, vf_hbm, srcp_hbm, dstp_hbm,
                  outwv_hbm, outz_hbm,
                  src0, src1, dq0, dq1, dr0, dr1, zr0, zr1, zc0, zc1,
                  k0, k1, q0, q1, v0, v1, zp0, zp1,
                  acc, accz,
                  si0, si1, sk0, sk1, sw0, sw1, sz0, sz1):
    src_v, dq_v, dr_v = (src0, src1), (dq0, dq1), (dr0, dr1)
    zr_v, zc_v = (zr0, zr1), (zc0, zc1)
    k_b, q_b, v_b, zp_b = (k0, k1), (q0, q1), (v0, v1), (zp0, zp1)
    sem_idx, sem_kqv = (si0, si1), (sk0, sk1)
    sem_wv, sem_z = (sw0, sw1), (sz0, sz1)
    cid = lax.axis_index("c")
    sid = lax.axis_index("s")
    zeros16 = jnp.zeros((GRP,), jnp.float32)
    iota16 = lax.iota(jnp.int32, GRP)
    ebase = sid * EPT

    for p in range(2):
        @pl.loop(0, CH)
        def _zrow(r, _p=p):
            for j in range(HALF // GRP):
                zp_b[_p][r, pl.ds(j * GRP, GRP)] = zeros16

    row0 = sid * ROWS_PT
    for j in range(ROWS_PT // CH):
        pltpu.sync_copy(zp_b[0], acc.at[pl.ds(row0 + j * CH, CH)])
    _rem = ROWS_PT % CH
    if _rem:
        pltpu.sync_copy(zp_b[0].at[pl.ds(0, _rem)],
                        acc.at[pl.ds(row0 + ROWS_PT - _rem, _rem)])
    zrow0 = sid * (ZROWS // NTILES)
    pltpu.sync_copy(zp_b[0], accz.at[pl.ds(zrow0, ZROWS // NTILES)])

    plsc.subcore_barrier()

    def idx_descr(ec, p):
        base = ebase + ec
        return (pltpu.make_async_copy(srcp_hbm.at[pl.ds(cid * E + base, CH)],
                                      src_v[p], sem_idx[p]),
                pltpu.make_async_copy(dstp_hbm.at[pl.ds(cid * E + base, CH)],
                                      dq_v[p], sem_idx[p]),
                pltpu.make_async_copy(dstp_hbm.at[pl.ds(base, CH)],
                                      dr_v[p], sem_idx[p]))

    def gather_descr(p):
        return (pltpu.make_async_copy(kf_hbm.at[src_v[p]], k_b[p],
                                      sem_kqv[p]),
                pltpu.make_async_copy(qf_hbm.at[dq_v[p]], q_b[p],
                                      sem_kqv[p]),
                pltpu.make_async_copy(vf_hbm.at[src_v[p]], v_b[p],
                                      sem_kqv[p]))

    def wv_start(p):
        pltpu.async_copy(v_b[p], acc.at[dr_v[p]], sem_wv[p], add=True)

    def wv_wait(p):
        pltpu.make_async_copy(v_b[p], acc.at[dr_v[p]], sem_wv[p]).wait()

    def z_start(p):
        pltpu.async_copy(zp_b[p], accz.at[zr_v[p]], sem_z[p], add=True)

    def z_wait(p):
        pltpu.make_async_copy(zp_b[p], accz.at[zr_v[p]], sem_z[p]).wait()

    def score_pass(p):
        @pl.loop(0, CH, step=GRP)
        def _grp(g):
            eids = g + iota16
            dvec = dr_v[p][pl.ds(g, GRP)]
            zr_v[p][pl.ds(g, GRP)] = lax.shift_right_logical(dvec, 5)
            zcol0 = jnp.bitwise_and(dvec, 31) * 4
            zc_v[p][pl.ds(g, GRP)] = zcol0
            for h in range(H // 2):
                acc_a = zeros16
                acc_b = zeros16
                for dd in range(0, DK, 2):
                    ca = jnp.full((GRP,), h * DK + dd, jnp.int32)
                    cb = jnp.full((GRP,), h * DK + dd + 1, jnp.int32)
                    acc_a = acc_a + (plsc.load_gather(k_b[p], [eids, ca])
                                     * plsc.load_gather(q_b[p], [eids, ca]))
                    acc_b = acc_b + (plsc.load_gather(k_b[p], [eids, cb])
                                     * plsc.load_gather(q_b[p], [eids, cb]))
                acc_s = acc_a + acc_b
                s = jnp.exp(jnp.clip(acc_s * _INV_SQRT_DK, -10.0, 10.0))
                plsc.store_scatter(zp_b[p], [eids, zcol0 + h], s)

    def clear_zpay(p):
        @pl.loop(0, CH, step=GRP)
        def _zclr(g):
            eids = g + iota16
            zcol0 = zc_v[p][pl.ds(g, GRP)]
            for h in range(H // 2):
                plsc.store_scatter(zp_b[p], [eids, zcol0 + h], zeros16)

    def payload_pass(p):
        @pl.loop(0, CH, step=GRP)
        def _grp2(g):
            eids = g + iota16
            zcol0 = zc_v[p][pl.ds(g, GRP)]
            for h in range(H // 2):
                s = plsc.load_gather(zp_b[p], [eids, zcol0 + h])
                for dd in range(DK):
                    col = jnp.full((GRP,), h * DK + dd, jnp.int32)
                    vv = plsc.load_gather(v_b[p], [eids, col])
                    plsc.store_scatter(v_b[p], [eids, col], vv * s)

    def body(ec, p, first=False, guard_next=False):
        q = 1 - p
        for d in gather_descr(p):
            d.wait()
        score_pass(p)
        if not first:
            z_wait(q)
            clear_zpay(q)
            wv_wait(q)
        def prefetch_idx():
            for d in idx_descr(ec + CH, q):
                d.start()
        if guard_next:
            pl.when(ec + CH < EPT)(prefetch_idx)
        else:
            prefetch_idx()
        payload_pass(p)
        wv_start(p)
        z_start(p)
        def issue_next():
            for d in idx_descr(ec + CH, q):
                d.wait()
            for d in gather_descr(q):
                d.start()
        if guard_next:
            pl.when(ec + CH < EPT)(issue_next)
        else:
            issue_next()

    for d in idx_descr(0, 0):
        d.start()
    for d in idx_descr(0, 0):
        d.wait()
    for d in gather_descr(0):
        d.start()
    body(0, 0, first=True)

    @pl.loop(CH, EPT, step=2 * CH)
    def _pair(e0):
        body(e0, 1)
        body(e0 + CH, 0, guard_next=True)

    z_wait(0)
    wv_wait(0)

    plsc.subcore_barrier()

    for j in range(ROWS_PT // CH):
        r = row0 + j * CH
        pltpu.sync_copy(acc.at[pl.ds(r, CH)], v_b[0])
        pltpu.sync_copy(v_b[0], outwv_hbm.at[pl.ds(cid * N_PAD + r, CH)])
    if _rem:
        r = row0 + ROWS_PT - _rem
        pltpu.sync_copy(acc.at[pl.ds(r, _rem)], v_b[0].at[pl.ds(0, _rem)])
        pltpu.sync_copy(v_b[0].at[pl.ds(0, _rem)],
                        outwv_hbm.at[pl.ds(cid * N_PAD + r, _rem)])
    pltpu.sync_copy(accz.at[pl.ds(zrow0, ZROWS // NTILES)], v_b[0])
    pltpu.sync_copy(v_b[0], outz_hbm.at[pl.ds(cid * ZROWS + zrow0,
                                              ZROWS // NTILES)])


def _edge_sc(kf, qf, vf, srcp, dstp):
    mesh = plsc.VectorSubcoreMesh(core_axis_name="c", subcore_axis_name="s")
    cp = pltpu.CompilerParams()
    if "needs_layout_passes" in pltpu.CompilerParams.__dataclass_fields__:
        cp = dataclasses.replace(cp, needs_layout_passes=False)
    kern = functools.partial(
        pl.kernel,
        mesh=mesh,
        compiler_params=cp,
        out_type=[jax.ShapeDtypeStruct((2 * N_PAD, HALF), jnp.float32),
                  jax.ShapeDtypeStruct((2 * ZROWS, HALF), jnp.float32)],
        scratch_types=(
            [pltpu.VMEM((CH,), jnp.int32)] * 10
            + [pltpu.VMEM((CH, HALF), jnp.float32)] * 8
            + [pltpu.VMEM_SHARED((N_PAD, HALF), jnp.float32),
               pltpu.VMEM_SHARED((ZROWS, HALF), jnp.float32)]
            + [pltpu.SemaphoreType.DMA] * 8
        ),
    )(_edge_sc_body)
    return kern(kf, qf, vf, srcp, dstp)


def _ln(x, g, b, eps=1e-5):
    m = jnp.mean(x, axis=-1, keepdims=True)
    v = jnp.mean((x - m) ** 2, axis=-1, keepdims=True)
    return (x - m) / jnp.sqrt(v + eps) * g + b


def _post_body(wv_ref, z_ref, x_ref, wo_ref, bo_ref, ln1g_ref, ln1b_ref,
               w1_ref, b1_ref, w2_ref, b2_ref, ln2g_ref, ln2b_ref, out_ref):
    wv = wv_ref[...]
    z = z_ref[...]
    zb = jnp.repeat(z, DK, axis=1)
    o = wv / (zb + 1e-12)
    x = x_ref[...]
    h1 = x + jnp.dot(o, wo_ref[...], preferred_element_type=jnp.float32) \
        + bo_ref[...]
    h1 = _ln(h1, ln1g_ref[...], ln1b_ref[...])
    h2 = jnp.maximum(
        jnp.dot(h1, w1_ref[...], preferred_element_type=jnp.float32)
        + b1_ref[...], 0.0)
    h3 = h1 + jnp.dot(h2, w2_ref[...], preferred_element_type=jnp.float32) \
        + b2_ref[...]
    out_ref[...] = _ln(h3, ln2g_ref[...], ln2b_ref[...])


def _post(wv, z, x, Wo, bo, ln1_g, ln1_b, W1, b1, W2, b2, ln2_g, ln2_b):
    grid = (N // ROW_BLK,)
    full = lambda r, c: pl.BlockSpec((r, c), lambda i: (0, 0))
    return pl.pallas_call(
        _post_body,
        grid=grid,
        in_specs=[
            pl.BlockSpec((ROW_BLK, NDIM), lambda i: (i, 0)),
            pl.BlockSpec((ROW_BLK, H), lambda i: (i, 0)),
            pl.BlockSpec((ROW_BLK, EDIM), lambda i: (i, 0)),
            full(NDIM, EDIM), full(1, EDIM), full(1, EDIM), full(1, EDIM),
            full(EDIM, 4 * EDIM), full(1, 4 * EDIM),
            full(4 * EDIM, EDIM), full(1, EDIM),
            full(1, EDIM), full(1, EDIM),
        ],
        out_specs=pl.BlockSpec((ROW_BLK, EDIM), lambda i: (i, 0)),
        out_shape=jax.ShapeDtypeStruct((N, EDIM), jnp.float32),
    )(wv, z, x, Wo, bo.reshape(1, EDIM), ln1_g.reshape(1, EDIM),
      ln1_b.reshape(1, EDIM), W1, b1.reshape(1, 4 * EDIM), W2,
      b2.reshape(1, EDIM), ln2_g.reshape(1, EDIM), ln2_b.reshape(1, EDIM))


def kernel(x, src_x, dst_x, edge_index, Wq, bq, Wk, Wv, Wo, bo, ln1_g, ln1_b,
           W1, b1, W2, b2, ln2_g, ln2_b):
    q3, k3, v3 = _qkv(x, src_x, dst_x, Wq, bq, Wk, Wv)
    qf = q3.reshape(2 * N, HALF)
    kf = k3.reshape(2 * N, HALF)
    vf = v3.reshape(2 * N, HALF)
    src = edge_index[0]
    dst = edge_index[1]
    off = jnp.array([[0], [N]], jnp.int32)
    srcp = (src[None, :] + off).reshape(2 * E)
    dstp = (dst[None, :] + off).reshape(2 * E)
    acc_wv, acc_z = _edge_sc(kf, qf, vf, srcp, dstp)
    wv = jnp.concatenate([acc_wv[:N], acc_wv[N_PAD:N_PAD + N]], axis=1)
    z0 = acc_z[:ZROWS].reshape(ZROWS * 32, 4)[:N]
    z1 = acc_z[ZROWS:].reshape(ZROWS * 32, 4)[:N]
    z = jnp.concatenate([z0, z1], axis=1)
    return _post(wv, z, x, Wo, bo, ln1_g, ln1_b, W1, b1, W2, b2,
                 ln2_g, ln2_b)


# speedup vs baseline: 18.8423x; 2.9154x over previous
import dataclasses
import functools
import math
import jax
import jax.numpy as jnp
from jax import lax
from jax.experimental import pallas as pl
from jax.experimental.pallas import tpu as pltpu
from jax.experimental.pallas import tpu_sc as plsc

N = 10000
E = 320000
EDIM = 128
NDIM = 256
H = 8
DK = NDIM // H

ROW_BLK = 1000
HALF = 128
CH = 32
GRP = 16
NTILES = 16
EPT = E // NTILES
N_PAD = 10112
ROWS_PT = N_PAD // NTILES
ZROWS = 512


def _qkv_body(x_ref, src_ref, dst_ref, wq_ref, bq_ref, wk_ref, wv_ref,
              q_ref, k_ref, v_ref):
    x = x_ref[...]
    q = jnp.dot(x, wq_ref[...], preferred_element_type=jnp.float32) \
        + bq_ref[...] + src_ref[...]
    k = jnp.dot(x, wk_ref[...], preferred_element_type=jnp.float32)
    v = jnp.dot(x, wv_ref[...], preferred_element_type=jnp.float32) \
        + dst_ref[...]
    q_ref[0], q_ref[1] = q[:, :HALF], q[:, HALF:]
    k_ref[0], k_ref[1] = k[:, :HALF], k[:, HALF:]
    v_ref[0], v_ref[1] = v[:, :HALF], v[:, HALF:]


def _qkv(x, src_x, dst_x, Wq, bq, Wk, Wv):
    grid = (N // ROW_BLK,)
    row_spec = pl.BlockSpec((ROW_BLK, EDIM), lambda i: (i, 0))
    row_spec_n = pl.BlockSpec((ROW_BLK, NDIM), lambda i: (i, 0))
    w_spec = pl.BlockSpec((EDIM, NDIM), lambda i: (0, 0))
    b_spec = pl.BlockSpec((1, NDIM), lambda i: (0, 0))
    out_spec = pl.BlockSpec((2, ROW_BLK, HALF), lambda i: (0, i, 0))
    out_sds = jax.ShapeDtypeStruct((2, N, HALF), jnp.float32)
    return pl.pallas_call(
        _qkv_body,
        grid=grid,
        in_specs=[row_spec, row_spec_n, row_spec_n, w_spec, b_spec, w_spec,
                  w_spec],
        out_specs=[out_spec, out_spec, out_spec],
        out_shape=[out_sds, out_sds, out_sds],
    )(x, src_x, dst_x, Wq, bq.reshape(1, NDIM), Wk, Wv)


_INV_SQRT_DK = 1.0 / math.sqrt(DK)


def _edge_sc_body(kf_hbm, qf_hbm, vf_hbm, srcp_hbm, dstp_hbm,
                  outwv_hbm, outz_hbm,
                  src0, src1, dq0, dq1, dr0, dr1, zr0, zr1, zc0, zc1,
                  sdr0, sdr1, szr0, szr1,
                  k0, k1, q0, q1, v0, v1, zp0, zp1,
                  acc, accz,
                  si0, si1, sk0, sk1, sw0, sw1, sz0, sz1):
    src_v, dq_v, dr_v = (src0, src1), (dq0, dq1), (dr0, dr1)
    zr_v, zc_v = (zr0, zr1), (zc0, zc1)
    sdr_v, szr_v = (sdr0, sdr1), (szr0, szr1)
    k_b, q_b, v_b, zp_b = (k0, k1), (q0, q1), (v0, v1), (zp0, zp1)
    sem_idx, sem_kqv = (si0, si1), (sk0, sk1)
    sem_wv, sem_z = (sw0, sw1), (sz0, sz1)
    cid = lax.axis_index("c")
    sid = lax.axis_index("s")
    zeros16 = jnp.zeros((GRP,), jnp.float32)
    iota16 = lax.iota(jnp.int32, GRP)
    ebase = sid * EPT

    for p in range(2):
        @pl.loop(0, CH)
        def _zrow(r, _p=p):
            for j in range(HALF // GRP):
                zp_b[_p][r, pl.ds(j * GRP, GRP)] = zeros16

    row0 = sid * ROWS_PT
    for j in range(ROWS_PT // CH):
        pltpu.sync_copy(zp_b[0], acc.at[pl.ds(row0 + j * CH, CH)])
    _rem = ROWS_PT % CH
    if _rem:
        pltpu.sync_copy(zp_b[0].at[pl.ds(0, _rem)],
                        acc.at[pl.ds(row0 + ROWS_PT - _rem, _rem)])
    zrow0 = sid * (ZROWS // NTILES)
    pltpu.sync_copy(zp_b[0], accz.at[pl.ds(zrow0, ZROWS // NTILES)])

    plsc.subcore_barrier()

    def idx_descr(ec, p):
        base = ebase + ec
        return (pltpu.make_async_copy(srcp_hbm.at[pl.ds(cid * E + base, CH)],
                                      src_v[p], sem_idx[p]),
                pltpu.make_async_copy(dstp_hbm.at[pl.ds(cid * E + base, CH)],
                                      dq_v[p], sem_idx[p]),
                pltpu.make_async_copy(dstp_hbm.at[pl.ds(base, CH)],
                                      dr_v[p], sem_idx[p]))

    def gather_descr(p):
        return (pltpu.make_async_copy(kf_hbm.at[src_v[p]], k_b[p],
                                      sem_kqv[p]),
                pltpu.make_async_copy(qf_hbm.at[dq_v[p]], q_b[p],
                                      sem_kqv[p]),
                pltpu.make_async_copy(vf_hbm.at[src_v[p]], v_b[p],
                                      sem_kqv[p]))

    def wv_start(p):
        pltpu.async_copy(v_b[p], acc.at[sdr_v[p]], sem_wv[p], add=True)

    def wv_wait(p):
        pltpu.make_async_copy(v_b[p], acc.at[sdr_v[p]], sem_wv[p]).wait()

    def z_start(p):
        pltpu.async_copy(zp_b[p], accz.at[szr_v[p]], sem_z[p], add=True)

    def z_wait(p):
        pltpu.make_async_copy(zp_b[p], accz.at[szr_v[p]], sem_z[p]).wait()

    zmask = iota16 < 4

    def compute_group(p, g):
        if True:
            dvec = dr_v[p][pl.ds(g, GRP)]
            zr_v[p][pl.ds(g, GRP)] = lax.shift_right_logical(dvec, 5)
            zcol0 = jnp.bitwise_and(dvec, 31) * 4
            zc_v[p][pl.ds(g, GRP)] = zcol0
            for i in range(GRP):
                e = g + i
                prods = [k_b[p][e, pl.ds(16 * j, 16)]
                         * q_b[p][e, pl.ds(16 * j, 16)]
                         for j in range(HALF // GRP)]
                svs = []
                for h in range(H // 2):
                    t = prods[2 * h] + prods[2 * h + 1]
                    s = jnp.sum(t) * _INV_SQRT_DK
                    sv = jnp.broadcast_to(s, (GRP,))
                    sv = jnp.exp(jnp.clip(sv, -10.0, 10.0))
                    svs.append(sv)
                    for j in (2 * h, 2 * h + 1):
                        v_b[p][e, pl.ds(16 * j, 16)] = \
                            v_b[p][e, pl.ds(16 * j, 16)] * sv
                sv_all = jnp.where(iota16 == 0, svs[0],
                                   jnp.where(iota16 == 1, svs[1],
                                             jnp.where(iota16 == 2, svs[2],
                                                       svs[3])))
                rows = jnp.broadcast_to(e, (GRP,))
                cols = jnp.broadcast_to(zcol0[i], (GRP,)) + iota16
                plsc.store_scatter(zp_b[p], [rows, cols], sv_all, mask=zmask)

    def clear_zpay(p):
        @pl.loop(0, CH, step=GRP)
        def _zclr(g):
            eids = g + iota16
            zcol0 = zc_v[p][pl.ds(g, GRP)]
            for h in range(H // 2):
                plsc.store_scatter(zp_b[p], [eids, zcol0 + h], zeros16)

    def body(ec, p, first=False, guard_next=False):
        q = 1 - p
        for d in gather_descr(p):
            d.wait()
        def prefetch_idx():
            for d in idx_descr(ec + CH, q):
                d.start()
        if guard_next:
            pl.when(ec + CH < EPT)(prefetch_idx)
        else:
            prefetch_idx()
        compute_group(p, 0)
        if not first:
            z_wait(q)
            clear_zpay(q)
            wv_wait(q)
        def issue_next():
            for d in idx_descr(ec + CH, q):
                d.wait()
            for d in gather_descr(q):
                d.start()
        if guard_next:
            pl.when(ec + CH < EPT)(issue_next)
        else:
            issue_next()
        compute_group(p, GRP)
        sdr_v[p][pl.ds(0, GRP)] = dr_v[p][pl.ds(0, GRP)]
        sdr_v[p][pl.ds(GRP, GRP)] = dr_v[p][pl.ds(GRP, GRP)]
        szr_v[p][pl.ds(0, GRP)] = zr_v[p][pl.ds(0, GRP)]
        szr_v[p][pl.ds(GRP, GRP)] = zr_v[p][pl.ds(GRP, GRP)]
        wv_start(p)
        z_start(p)

    for d in idx_descr(0, 0):
        d.start()
    for d in idx_descr(0, 0):
        d.wait()
    for d in gather_descr(0):
        d.start()
    body(0, 0, first=True)

    @pl.loop(CH, EPT, step=2 * CH)
    def _pair(e0):
        body(e0, 1)
        body(e0 + CH, 0, guard_next=True)

    z_wait(0)
    wv_wait(0)

    plsc.subcore_barrier()

    for j in range(ROWS_PT // CH):
        r = row0 + j * CH
        pltpu.sync_copy(acc.at[pl.ds(r, CH)], v_b[0])
        pltpu.sync_copy(v_b[0], outwv_hbm.at[pl.ds(cid * N_PAD + r, CH)])
    if _rem:
        r = row0 + ROWS_PT - _rem
        pltpu.sync_copy(acc.at[pl.ds(r, _rem)], v_b[0].at[pl.ds(0, _rem)])
        pltpu.sync_copy(v_b[0].at[pl.ds(0, _rem)],
                        outwv_hbm.at[pl.ds(cid * N_PAD + r, _rem)])
    pltpu.sync_copy(accz.at[pl.ds(zrow0, ZROWS // NTILES)], v_b[0])
    pltpu.sync_copy(v_b[0], outz_hbm.at[pl.ds(cid * ZROWS + zrow0,
                                              ZROWS // NTILES)])


def _edge_sc(kf, qf, vf, srcp, dstp):
    mesh = plsc.VectorSubcoreMesh(core_axis_name="c", subcore_axis_name="s")
    cp = pltpu.CompilerParams()
    if "needs_layout_passes" in pltpu.CompilerParams.__dataclass_fields__:
        cp = dataclasses.replace(cp, needs_layout_passes=False)
    kern = functools.partial(
        pl.kernel,
        mesh=mesh,
        compiler_params=cp,
        out_type=[jax.ShapeDtypeStruct((2 * N_PAD, HALF), jnp.float32),
                  jax.ShapeDtypeStruct((2 * ZROWS, HALF), jnp.float32)],
        scratch_types=(
            [pltpu.VMEM((CH,), jnp.int32)] * 14
            + [pltpu.VMEM((CH, HALF), jnp.float32)] * 8
            + [pltpu.VMEM_SHARED((N_PAD, HALF), jnp.float32),
               pltpu.VMEM_SHARED((ZROWS, HALF), jnp.float32)]
            + [pltpu.SemaphoreType.DMA] * 8
        ),
    )(_edge_sc_body)
    return kern(kf, qf, vf, srcp, dstp)


def _ln(x, g, b, eps=1e-5):
    m = jnp.mean(x, axis=-1, keepdims=True)
    v = jnp.mean((x - m) ** 2, axis=-1, keepdims=True)
    return (x - m) / jnp.sqrt(v + eps) * g + b


def _post_body(wv_ref, z_ref, x_ref, wo_ref, bo_ref, ln1g_ref, ln1b_ref,
               w1_ref, b1_ref, w2_ref, b2_ref, ln2g_ref, ln2b_ref, out_ref):
    wv = wv_ref[...]
    z = z_ref[...]
    zb = jnp.repeat(z, DK, axis=1)
    o = wv / (zb + 1e-12)
    x = x_ref[...]
    h1 = x + jnp.dot(o, wo_ref[...], preferred_element_type=jnp.float32) \
        + bo_ref[...]
    h1 = _ln(h1, ln1g_ref[...], ln1b_ref[...])
    h2 = jnp.maximum(
        jnp.dot(h1, w1_ref[...], preferred_element_type=jnp.float32)
        + b1_ref[...], 0.0)
    h3 = h1 + jnp.dot(h2, w2_ref[...], preferred_element_type=jnp.float32) \
        + b2_ref[...]
    out_ref[...] = _ln(h3, ln2g_ref[...], ln2b_ref[...])


def _post(wv, z, x, Wo, bo, ln1_g, ln1_b, W1, b1, W2, b2, ln2_g, ln2_b):
    grid = (N // ROW_BLK,)
    full = lambda r, c: pl.BlockSpec((r, c), lambda i: (0, 0))
    return pl.pallas_call(
        _post_body,
        grid=grid,
        in_specs=[
            pl.BlockSpec((ROW_BLK, NDIM), lambda i: (i, 0)),
            pl.BlockSpec((ROW_BLK, H), lambda i: (i, 0)),
            pl.BlockSpec((ROW_BLK, EDIM), lambda i: (i, 0)),
            full(NDIM, EDIM), full(1, EDIM), full(1, EDIM), full(1, EDIM),
            full(EDIM, 4 * EDIM), full(1, 4 * EDIM),
            full(4 * EDIM, EDIM), full(1, EDIM),
            full(1, EDIM), full(1, EDIM),
        ],
        out_specs=pl.BlockSpec((ROW_BLK, EDIM), lambda i: (i, 0)),
        out_shape=jax.ShapeDtypeStruct((N, EDIM), jnp.float32),
    )(wv, z, x, Wo, bo.reshape(1, EDIM), ln1_g.reshape(1, EDIM),
      ln1_b.reshape(1, EDIM), W1, b1.reshape(1, 4 * EDIM), W2,
      b2.reshape(1, EDIM), ln2_g.reshape(1, EDIM), ln2_b.reshape(1, EDIM))


def kernel(x, src_x, dst_x, edge_index, Wq, bq, Wk, Wv, Wo, bo, ln1_g, ln1_b,
           W1, b1, W2, b2, ln2_g, ln2_b):
    q3, k3, v3 = _qkv(x, src_x, dst_x, Wq, bq, Wk, Wv)
    qf = q3.reshape(2 * N, HALF)
    kf = k3.reshape(2 * N, HALF)
    vf = v3.reshape(2 * N, HALF)
    src = edge_index[0]
    dst = edge_index[1]
    off = jnp.array([[0], [N]], jnp.int32)
    srcp = (src[None, :] + off).reshape(2 * E)
    dstp = (dst[None, :] + off).reshape(2 * E)
    acc_wv, acc_z = _edge_sc(kf, qf, vf, srcp, dstp)
    wv = jnp.concatenate([acc_wv[:N], acc_wv[N_PAD:N_PAD + N]], axis=1)
    z0 = acc_z[:ZROWS].reshape(ZROWS * 32, 4)[:N]
    z1 = acc_z[ZROWS:].reshape(ZROWS * 32, 4)[:N]
    z = jnp.concatenate([z0, z1], axis=1)
    return _post(wv, z, x, Wo, bo, ln1_g, ln1_b, W1, b1, W2, b2,
                 ln2_g, ln2_b)


# speedup vs baseline: 18.9775x; 1.0072x over previous
import dataclasses
import functools
import math
import jax
import jax.numpy as jnp
from jax import lax
from jax.experimental import pallas as pl
from jax.experimental.pallas import tpu as pltpu
from jax.experimental.pallas import tpu_sc as plsc

N = 10000
E = 320000
EDIM = 128
NDIM = 256
H = 8
DK = NDIM // H

ROW_BLK = 1000
HALF = 128
CH = 32
GRP = 16
NTILES = 16
EPT = E // NTILES
N_PAD = 10112
ROWS_PT = N_PAD // NTILES
ZROWS = 512
_INV_SQRT_DK = 1.0 / math.sqrt(DK)


def _qkv_body(x_ref, src_ref, dst_ref, wq_ref, bq_ref, wk_ref, wv_ref,
              q_ref, k_ref, v_ref):
    x = x_ref[...]
    q = jnp.dot(x, wq_ref[...], preferred_element_type=jnp.float32) \
        + bq_ref[...] + src_ref[...]
    k = jnp.dot(x, wk_ref[...],
                preferred_element_type=jnp.float32) * _INV_SQRT_DK
    v = jnp.dot(x, wv_ref[...], preferred_element_type=jnp.float32) \
        + dst_ref[...]
    q_ref[0], q_ref[1] = q[:, :HALF], q[:, HALF:]
    k_ref[0], k_ref[1] = k[:, :HALF], k[:, HALF:]
    v_ref[0], v_ref[1] = v[:, :HALF], v[:, HALF:]


def _qkv(x, src_x, dst_x, Wq, bq, Wk, Wv):
    grid = (N // ROW_BLK,)
    row_spec = pl.BlockSpec((ROW_BLK, EDIM), lambda i: (i, 0))
    row_spec_n = pl.BlockSpec((ROW_BLK, NDIM), lambda i: (i, 0))
    w_spec = pl.BlockSpec((EDIM, NDIM), lambda i: (0, 0))
    b_spec = pl.BlockSpec((1, NDIM), lambda i: (0, 0))
    out_spec = pl.BlockSpec((2, ROW_BLK, HALF), lambda i: (0, i, 0))
    out_sds = jax.ShapeDtypeStruct((2, N, HALF), jnp.float32)
    return pl.pallas_call(
        _qkv_body,
        grid=grid,
        in_specs=[row_spec, row_spec_n, row_spec_n, w_spec, b_spec, w_spec,
                  w_spec],
        out_specs=[out_spec, out_spec, out_spec],
        out_shape=[out_sds, out_sds, out_sds],
    )(x, src_x, dst_x, Wq, bq.reshape(1, NDIM), Wk, Wv)


_GDN = lax.GatherDimensionNumbers(offset_dims=(), collapsed_slice_dims=(0,),
                                  start_index_map=(0,))


def _lane_shuffle(x, idx):
    return lax.gather(x, idx[:, None], _GDN, (1,),
                      mode=lax.GatherScatterMode.PROMISE_IN_BOUNDS)


def _edge_sc_body(kf_hbm, qf_hbm, vf_hbm, srcp_hbm, dstp_hbm,
                  outwv_hbm, outz_hbm,
                  src0, src1, dq0, dq1, dr0, dr1, zr0, zr1, zc0, zc1,
                  sdr0, sdr1, szr0, szr1,
                  k0, k1, q0, q1, v0, v1, zp0, zp1,
                  acc, accz,
                  si0, si1, sk0, sk1, sw0, sw1, sz0, sz1):
    src_v, dq_v, dr_v = (src0, src1), (dq0, dq1), (dr0, dr1)
    zr_v, zc_v = (zr0, zr1), (zc0, zc1)
    sdr_v, szr_v = (sdr0, sdr1), (szr0, szr1)
    k_b, q_b, v_b, zp_b = (k0, k1), (q0, q1), (v0, v1), (zp0, zp1)
    sem_idx, sem_kqv = (si0, si1), (sk0, sk1)
    sem_wv, sem_z = (sw0, sw1), (sz0, sz1)
    cid = lax.axis_index("c")
    sid = lax.axis_index("s")
    zeros16 = jnp.zeros((GRP,), jnp.float32)
    iota16 = lax.iota(jnp.int32, GRP)
    ebase = sid * EPT

    for p in range(2):
        @pl.loop(0, CH)
        def _zrow(r, _p=p):
            for j in range(HALF // GRP):
                zp_b[_p][r, pl.ds(j * GRP, GRP)] = zeros16

    row0 = sid * ROWS_PT
    for j in range(ROWS_PT // CH):
        pltpu.sync_copy(zp_b[0], acc.at[pl.ds(row0 + j * CH, CH)])
    _rem = ROWS_PT % CH
    if _rem:
        pltpu.sync_copy(zp_b[0].at[pl.ds(0, _rem)],
                        acc.at[pl.ds(row0 + ROWS_PT - _rem, _rem)])
    zrow0 = sid * (ZROWS // NTILES)
    pltpu.sync_copy(zp_b[0], accz.at[pl.ds(zrow0, ZROWS // NTILES)])

    plsc.subcore_barrier()

    def idx_descr(ec, p):
        base = ebase + ec
        return (pltpu.make_async_copy(srcp_hbm.at[pl.ds(cid * E + base, CH)],
                                      src_v[p], sem_idx[p]),
                pltpu.make_async_copy(dstp_hbm.at[pl.ds(cid * E + base, CH)],
                                      dq_v[p], sem_idx[p]),
                pltpu.make_async_copy(dstp_hbm.at[pl.ds(base, CH)],
                                      dr_v[p], sem_idx[p]))

    def gather_descr(p):
        return (pltpu.make_async_copy(kf_hbm.at[src_v[p]], k_b[p],
                                      sem_kqv[p]),
                pltpu.make_async_copy(qf_hbm.at[dq_v[p]], q_b[p],
                                      sem_kqv[p]),
                pltpu.make_async_copy(vf_hbm.at[src_v[p]], v_b[p],
                                      sem_kqv[p]))

    def wv_start(p):
        pltpu.async_copy(v_b[p], acc.at[sdr_v[p]], sem_wv[p], add=True)

    def wv_wait(p):
        pltpu.make_async_copy(v_b[p], acc.at[sdr_v[p]], sem_wv[p]).wait()

    def z_start(p):
        pltpu.async_copy(zp_b[p], accz.at[szr_v[p]], sem_z[p], add=True)

    def z_wait(p):
        pltpu.make_async_copy(zp_b[p], accz.at[szr_v[p]], sem_z[p]).wait()

    zmask = iota16 < 4
    perms = [iota16 ^ kk for kk in (1, 2, 4, 8)]

    def compute_group(p, g):
        if True:
            dvec = dr_v[p][pl.ds(g, GRP)]
            zr_v[p][pl.ds(g, GRP)] = lax.shift_right_logical(dvec, 5)
            zcol0 = jnp.bitwise_and(dvec, 31) * 4
            zc_v[p][pl.ds(g, GRP)] = zcol0
            for i in range(GRP):
                e = g + i
                prods = [k_b[p][e, pl.ds(16 * j, 16)]
                         * q_b[p][e, pl.ds(16 * j, 16)]
                         for j in range(HALF // GRP)]
                svs = []
                for h in range(H // 2):
                    t = prods[2 * h] + prods[2 * h + 1]
                    for pm in perms:
                        t = t + _lane_shuffle(t, pm)
                    sv = jnp.exp(jnp.clip(t, -10.0, 10.0))
                    svs.append(sv)
                    for j in (2 * h, 2 * h + 1):
                        v_b[p][e, pl.ds(16 * j, 16)] = \
                            v_b[p][e, pl.ds(16 * j, 16)] * sv
                sv_all = jnp.where(iota16 == 0, svs[0],
                                   jnp.where(iota16 == 1, svs[1],
                                             jnp.where(iota16 == 2, svs[2],
                                                       svs[3])))
                rows = jnp.broadcast_to(e, (GRP,))
                cols = jnp.broadcast_to(zcol0[i], (GRP,)) + iota16
                plsc.store_scatter(zp_b[p], [rows, cols], sv_all, mask=zmask)

    def clear_zpay(p):
        @pl.loop(0, CH, step=GRP)
        def _zclr(g):
            eids = g + iota16
            zcol0 = zc_v[p][pl.ds(g, GRP)]
            for h in range(H // 2):
                plsc.store_scatter(zp_b[p], [eids, zcol0 + h], zeros16)

    def body(ec, p, first=False, guard_next=False):
        q = 1 - p
        for d in gather_descr(p):
            d.wait()
        def prefetch_idx():
            for d in idx_descr(ec + CH, q):
                d.start()
        if guard_next:
            pl.when(ec + CH < EPT)(prefetch_idx)
        else:
            prefetch_idx()
        compute_group(p, 0)
        if not first:
            z_wait(q)
            clear_zpay(q)
            wv_wait(q)
        def issue_next():
            for d in idx_descr(ec + CH, q):
                d.wait()
            for d in gather_descr(q):
                d.start()
        if guard_next:
            pl.when(ec + CH < EPT)(issue_next)
        else:
            issue_next()
        compute_group(p, GRP)
        sdr_v[p][pl.ds(0, GRP)] = dr_v[p][pl.ds(0, GRP)]
        sdr_v[p][pl.ds(GRP, GRP)] = dr_v[p][pl.ds(GRP, GRP)]
        szr_v[p][pl.ds(0, GRP)] = zr_v[p][pl.ds(0, GRP)]
        szr_v[p][pl.ds(GRP, GRP)] = zr_v[p][pl.ds(GRP, GRP)]
        wv_start(p)
        z_start(p)

    for d in idx_descr(0, 0):
        d.start()
    for d in idx_descr(0, 0):
        d.wait()
    for d in gather_descr(0):
        d.start()
    body(0, 0, first=True)

    @pl.loop(CH, EPT, step=2 * CH)
    def _pair(e0):
        body(e0, 1)
        body(e0 + CH, 0, guard_next=True)

    z_wait(0)
    wv_wait(0)

    plsc.subcore_barrier()

    for j in range(ROWS_PT // CH):
        r = row0 + j * CH
        pltpu.sync_copy(acc.at[pl.ds(r, CH)], v_b[0])
        pltpu.sync_copy(v_b[0], outwv_hbm.at[pl.ds(cid * N_PAD + r, CH)])
    if _rem:
        r = row0 + ROWS_PT - _rem
        pltpu.sync_copy(acc.at[pl.ds(r, _rem)], v_b[0].at[pl.ds(0, _rem)])
        pltpu.sync_copy(v_b[0].at[pl.ds(0, _rem)],
                        outwv_hbm.at[pl.ds(cid * N_PAD + r, _rem)])
    pltpu.sync_copy(accz.at[pl.ds(zrow0, ZROWS // NTILES)], v_b[0])
    pltpu.sync_copy(v_b[0], outz_hbm.at[pl.ds(cid * ZROWS + zrow0,
                                              ZROWS // NTILES)])


def _edge_sc(kf, qf, vf, srcp, dstp):
    mesh = plsc.VectorSubcoreMesh(core_axis_name="c", subcore_axis_name="s")
    cp = pltpu.CompilerParams()
    if "needs_layout_passes" in pltpu.CompilerParams.__dataclass_fields__:
        cp = dataclasses.replace(cp, needs_layout_passes=False)
    kern = functools.partial(
        pl.kernel,
        mesh=mesh,
        compiler_params=cp,
        out_type=[jax.ShapeDtypeStruct((2 * N_PAD, HALF), jnp.float32),
                  jax.ShapeDtypeStruct((2 * ZROWS, HALF), jnp.float32)],
        scratch_types=(
            [pltpu.VMEM((CH,), jnp.int32)] * 14
            + [pltpu.VMEM((CH, HALF), jnp.float32)] * 8
            + [pltpu.VMEM_SHARED((N_PAD, HALF), jnp.float32),
               pltpu.VMEM_SHARED((ZROWS, HALF), jnp.float32)]
            + [pltpu.SemaphoreType.DMA] * 8
        ),
    )(_edge_sc_body)
    return kern(kf, qf, vf, srcp, dstp)


def _ln(x, g, b, eps=1e-5):
    m = jnp.mean(x, axis=-1, keepdims=True)
    v = jnp.mean((x - m) ** 2, axis=-1, keepdims=True)
    return (x - m) / jnp.sqrt(v + eps) * g + b


def _post_body(wv_ref, z_ref, x_ref, wo_ref, bo_ref, ln1g_ref, ln1b_ref,
               w1_ref, b1_ref, w2_ref, b2_ref, ln2g_ref, ln2b_ref, out_ref):
    wv = wv_ref[...]
    z = z_ref[...]
    zb = jnp.repeat(z, DK, axis=1)
    o = wv / (zb + 1e-12)
    x = x_ref[...]
    h1 = x + jnp.dot(o, wo_ref[...], preferred_element_type=jnp.float32) \
        + bo_ref[...]
    h1 = _ln(h1, ln1g_ref[...], ln1b_ref[...])
    h2 = jnp.maximum(
        jnp.dot(h1, w1_ref[...], preferred_element_type=jnp.float32)
        + b1_ref[...], 0.0)
    h3 = h1 + jnp.dot(h2, w2_ref[...], preferred_element_type=jnp.float32) \
        + b2_ref[...]
    out_ref[...] = _ln(h3, ln2g_ref[...], ln2b_ref[...])


def _post(wv, z, x, Wo, bo, ln1_g, ln1_b, W1, b1, W2, b2, ln2_g, ln2_b):
    grid = (N // ROW_BLK,)
    full = lambda r, c: pl.BlockSpec((r, c), lambda i: (0, 0))
    return pl.pallas_call(
        _post_body,
        grid=grid,
        in_specs=[
            pl.BlockSpec((ROW_BLK, NDIM), lambda i: (i, 0)),
            pl.BlockSpec((ROW_BLK, H), lambda i: (i, 0)),
            pl.BlockSpec((ROW_BLK, EDIM), lambda i: (i, 0)),
            full(NDIM, EDIM), full(1, EDIM), full(1, EDIM), full(1, EDIM),
            full(EDIM, 4 * EDIM), full(1, 4 * EDIM),
            full(4 * EDIM, EDIM), full(1, EDIM),
            full(1, EDIM), full(1, EDIM),
        ],
        out_specs=pl.BlockSpec((ROW_BLK, EDIM), lambda i: (i, 0)),
        out_shape=jax.ShapeDtypeStruct((N, EDIM), jnp.float32),
    )(wv, z, x, Wo, bo.reshape(1, EDIM), ln1_g.reshape(1, EDIM),
      ln1_b.reshape(1, EDIM), W1, b1.reshape(1, 4 * EDIM), W2,
      b2.reshape(1, EDIM), ln2_g.reshape(1, EDIM), ln2_b.reshape(1, EDIM))


def kernel(x, src_x, dst_x, edge_index, Wq, bq, Wk, Wv, Wo, bo, ln1_g, ln1_b,
           W1, b1, W2, b2, ln2_g, ln2_b):
    q3, k3, v3 = _qkv(x, src_x, dst_x, Wq, bq, Wk, Wv)
    qf = q3.reshape(2 * N, HALF)
    kf = k3.reshape(2 * N, HALF)
    vf = v3.reshape(2 * N, HALF)
    src = edge_index[0]
    dst = edge_index[1]
    off = jnp.array([[0], [N]], jnp.int32)
    srcp = (src[None, :] + off).reshape(2 * E)
    dstp = (dst[None, :] + off).reshape(2 * E)
    acc_wv, acc_z = _edge_sc(kf, qf, vf, srcp, dstp)
    wv = jnp.concatenate([acc_wv[:N], acc_wv[N_PAD:N_PAD + N]], axis=1)
    z0 = acc_z[:ZROWS].reshape(ZROWS * 32, 4)[:N]
    z1 = acc_z[ZROWS:].reshape(ZROWS * 32, 4)[:N]
    z = jnp.concatenate([z0, z1], axis=1)
    return _post(wv, z, x, Wo, bo, ln1_g, ln1_b, W1, b1, W2, b2,
                 ln2_g, ln2_b)


# speedup vs baseline: 19.1534x; 1.0093x over previous
import dataclasses
import functools
import math
import jax
import jax.numpy as jnp
from jax import lax
from jax.experimental import pallas as pl
from jax.experimental.pallas import tpu as pltpu
from jax.experimental.pallas import tpu_sc as plsc

N = 10000
E = 320000
EDIM = 128
NDIM = 256
H = 8
DK = NDIM // H

ROW_BLK = 1000
HALF = 128
CH = 32
GRP = 16
NTILES = 16
EPT = E // NTILES
N_PAD = 10112
ZROWS = 512
ACC_ROWS = N_PAD + ZROWS
ROWS_PT = ACC_ROWS // NTILES
_INV_SQRT_DK = 1.0 / math.sqrt(DK)


def _qkv_body(x_ref, src_ref, dst_ref, wq_ref, bq_ref, wk_ref, wv_ref,
              q_ref, k_ref, v_ref):
    x = x_ref[...]
    q = jnp.dot(x, wq_ref[...], preferred_element_type=jnp.float32) \
        + bq_ref[...] + src_ref[...]
    k = jnp.dot(x, wk_ref[...],
                preferred_element_type=jnp.float32) * _INV_SQRT_DK
    v = jnp.dot(x, wv_ref[...], preferred_element_type=jnp.float32) \
        + dst_ref[...]
    q_ref[0], q_ref[1] = q[:, :HALF], q[:, HALF:]
    k_ref[0], k_ref[1] = k[:, :HALF], k[:, HALF:]
    v_ref[0], v_ref[1] = v[:, :HALF], v[:, HALF:]


def _qkv(x, src_x, dst_x, Wq, bq, Wk, Wv):
    grid = (N // ROW_BLK,)
    row_spec = pl.BlockSpec((ROW_BLK, EDIM), lambda i: (i, 0))
    row_spec_n = pl.BlockSpec((ROW_BLK, NDIM), lambda i: (i, 0))
    w_spec = pl.BlockSpec((EDIM, NDIM), lambda i: (0, 0))
    b_spec = pl.BlockSpec((1, NDIM), lambda i: (0, 0))
    out_spec = pl.BlockSpec((2, ROW_BLK, HALF), lambda i: (0, i, 0))
    out_sds = jax.ShapeDtypeStruct((2, N, HALF), jnp.float32)
    return pl.pallas_call(
        _qkv_body,
        grid=grid,
        in_specs=[row_spec, row_spec_n, row_spec_n, w_spec, b_spec, w_spec,
                  w_spec],
        out_specs=[out_spec, out_spec, out_spec],
        out_shape=[out_sds, out_sds, out_sds],
    )(x, src_x, dst_x, Wq, bq.reshape(1, NDIM), Wk, Wv)


_GDN = lax.GatherDimensionNumbers(offset_dims=(), collapsed_slice_dims=(0,),
                                  start_index_map=(0,))


def _lane_shuffle(x, idx):
    return lax.gather(x, idx[:, None], _GDN, (1,),
                      mode=lax.GatherScatterMode.PROMISE_IN_BOUNDS)


def _edge_sc_body(kf_hbm, qf_hbm, vf_hbm, comb_hbm, out_hbm,
                  idx0, idx1, sidx0, sidx1, szc0, szc1,
                  k0, k1, q0, q1, v0, v1, pay0, pay1,
                  acc,
                  si0, si1, sk0, sk1, ss0, ss1):
    idx_b, sidx_b, szc_v = (idx0, idx1), (sidx0, sidx1), (szc0, szc1)
    k_b, q_b, v_b, pay_b = (k0, k1), (q0, q1), (v0, v1), (pay0, pay1)
    sem_idx, sem_kqv, sem_sc = (si0, si1), (sk0, sk1), (ss0, ss1)
    cid = lax.axis_index("c")
    sid = lax.axis_index("s")
    zeros16 = jnp.zeros((GRP,), jnp.float32)
    iota16 = lax.iota(jnp.int32, GRP)
    nchunks = EPT // CH

    @pl.loop(0, 2 * CH)
    def _zrow(r):
        for j in range(HALF // GRP):
            pay_b[0][r, pl.ds(j * GRP, GRP)] = zeros16
            pay_b[1][r, pl.ds(j * GRP, GRP)] = zeros16

    row0 = sid * ROWS_PT
    for j in range(ROWS_PT // (2 * CH)):
        pltpu.sync_copy(pay_b[0], acc.at[pl.ds(row0 + j * 2 * CH, 2 * CH)])
    _rem = ROWS_PT % (2 * CH)
    if _rem:
        pltpu.sync_copy(pay_b[0].at[pl.ds(0, _rem)],
                        acc.at[pl.ds(row0 + ROWS_PT - _rem, _rem)])

    plsc.subcore_barrier()

    def idx_descr(ec, p):
        off = (cid * NTILES * nchunks + sid * nchunks + ec // CH) * 3 * CH
        return pltpu.make_async_copy(comb_hbm.at[pl.ds(off, 3 * CH)],
                                     idx_b[p], sem_idx[p])

    def gather_descr(p):
        src_ref = idx_b[p].at[pl.ds(0, CH)]
        dq_ref = idx_b[p].at[pl.ds(CH, CH)]
        return (pltpu.make_async_copy(kf_hbm.at[src_ref], k_b[p],
                                      sem_kqv[p]),
                pltpu.make_async_copy(qf_hbm.at[dq_ref], q_b[p],
                                      sem_kqv[p]),
                pltpu.make_async_copy(vf_hbm.at[src_ref], v_b[p],
                                      sem_kqv[p]))

    def sc_start(p):
        pltpu.async_copy(pay_b[p], acc.at[sidx_b[p]], sem_sc[p], add=True)

    def sc_wait(p):
        pltpu.make_async_copy(pay_b[p], acc.at[sidx_b[p]], sem_sc[p]).wait()

    zmask = iota16 < 4
    perms = [iota16 ^ kk for kk in (1, 2, 4, 8)]

    def compute_group(p, g):
        dvec = idx_b[p][pl.ds(2 * CH + g, GRP)]
        sidx_b[p][pl.ds(g, GRP)] = dvec
        sidx_b[p][pl.ds(CH + g, GRP)] = \
            N_PAD + lax.shift_right_logical(dvec, 5)
        zcol0 = jnp.bitwise_and(dvec, 31) * 4
        szc_v[p][pl.ds(g, GRP)] = zcol0
        for i in range(GRP):
            e = g + i
            prods = [k_b[p][e, pl.ds(16 * j, 16)]
                     * q_b[p][e, pl.ds(16 * j, 16)]
                     for j in range(HALF // GRP)]
            svs = []
            for h in range(H // 2):
                t = prods[2 * h] + prods[2 * h + 1]
                for pm in perms:
                    t = t + _lane_shuffle(t, pm)
                sv = jnp.exp(jnp.clip(t, -10.0, 10.0))
                svs.append(sv)
                for j in (2 * h, 2 * h + 1):
                    pay_b[p][e, pl.ds(16 * j, 16)] = \
                        v_b[p][e, pl.ds(16 * j, 16)] * sv
            sv_all = jnp.where(iota16 == 0, svs[0],
                               jnp.where(iota16 == 1, svs[1],
                                         jnp.where(iota16 == 2, svs[2],
                                                   svs[3])))
            rows = jnp.broadcast_to(CH + e, (GRP,))
            cols = jnp.broadcast_to(zcol0[i], (GRP,)) + iota16
            plsc.store_scatter(pay_b[p], [rows, cols], sv_all, mask=zmask)

    def clear_zpay(p):
        for g in range(0, CH, GRP):
            eids = CH + g + iota16
            zcol0 = szc_v[p][pl.ds(g, GRP)]
            for h in range(H // 2):
                plsc.store_scatter(pay_b[p], [eids, zcol0 + h], zeros16)

    def body(ec, p, first=False, guard_next=False):
        q = 1 - p
        for d in gather_descr(p):
            d.wait()
        def prefetch_idx():
            idx_descr(ec + CH, q).start()
        if guard_next:
            pl.when(ec + CH < EPT)(prefetch_idx)
        else:
            prefetch_idx()
        compute_group(p, 0)
        if not first:
            sc_wait(q)
            clear_zpay(q)
        def issue_next():
            idx_descr(ec + CH, q).wait()
            for d in gather_descr(q):
                d.start()
        if guard_next:
            pl.when(ec + CH < EPT)(issue_next)
        else:
            issue_next()
        compute_group(p, GRP)
        sc_start(p)

    idx_descr(0, 0).start()
    idx_descr(0, 0).wait()
    for d in gather_descr(0):
        d.start()
    body(0, 0, first=True)

    @pl.loop(CH, EPT, step=2 * CH)
    def _pair(e0):
        body(e0, 1)
        body(e0 + CH, 0, guard_next=True)

    sc_wait(0)

    plsc.subcore_barrier()

    for j in range(ROWS_PT // (2 * CH)):
        r = row0 + j * 2 * CH
        pltpu.sync_copy(acc.at[pl.ds(r, 2 * CH)], pay_b[0])
        pltpu.sync_copy(pay_b[0],
                        out_hbm.at[pl.ds(cid * ACC_ROWS + r, 2 * CH)])
    if _rem:
        r = row0 + ROWS_PT - _rem
        pltpu.sync_copy(acc.at[pl.ds(r, _rem)], pay_b[0].at[pl.ds(0, _rem)])
        pltpu.sync_copy(pay_b[0].at[pl.ds(0, _rem)],
                        out_hbm.at[pl.ds(cid * ACC_ROWS + r, _rem)])


def _edge_sc(kf, qf, vf, comb):
    mesh = plsc.VectorSubcoreMesh(core_axis_name="c", subcore_axis_name="s")
    cp = pltpu.CompilerParams()
    if "needs_layout_passes" in pltpu.CompilerParams.__dataclass_fields__:
        cp = dataclasses.replace(cp, needs_layout_passes=False)
    kern = functools.partial(
        pl.kernel,
        mesh=mesh,
        compiler_params=cp,
        out_type=jax.ShapeDtypeStruct((2 * ACC_ROWS, HALF), jnp.float32),
        scratch_types=(
            [pltpu.VMEM((3 * CH,), jnp.int32)] * 2
            + [pltpu.VMEM((2 * CH,), jnp.int32)] * 2
            + [pltpu.VMEM((CH,), jnp.int32)] * 2
            + [pltpu.VMEM((CH, HALF), jnp.float32)] * 6
            + [pltpu.VMEM((2 * CH, HALF), jnp.float32)] * 2
            + [pltpu.VMEM_SHARED((ACC_ROWS, HALF), jnp.float32)]
            + [pltpu.SemaphoreType.DMA] * 6
        ),
    )(_edge_sc_body)
    return kern(kf, qf, vf, comb)


def _ln(x, g, b, eps=1e-5):
    m = jnp.mean(x, axis=-1, keepdims=True)
    v = jnp.mean((x - m) ** 2, axis=-1, keepdims=True)
    return (x - m) / jnp.sqrt(v + eps) * g + b


def _post_body(wv_ref, z_ref, x_ref, wo_ref, bo_ref, ln1g_ref, ln1b_ref,
               w1_ref, b1_ref, w2_ref, b2_ref, ln2g_ref, ln2b_ref, out_ref):
    wv = wv_ref[...]
    z = z_ref[...]
    zb = jnp.repeat(z, DK, axis=1)
    o = wv / (zb + 1e-12)
    x = x_ref[...]
    h1 = x + jnp.dot(o, wo_ref[...], preferred_element_type=jnp.float32) \
        + bo_ref[...]
    h1 = _ln(h1, ln1g_ref[...], ln1b_ref[...])
    h2 = jnp.maximum(
        jnp.dot(h1, w1_ref[...], preferred_element_type=jnp.float32)
        + b1_ref[...], 0.0)
    h3 = h1 + jnp.dot(h2, w2_ref[...], preferred_element_type=jnp.float32) \
        + b2_ref[...]
    out_ref[...] = _ln(h3, ln2g_ref[...], ln2b_ref[...])


def _post(wv, z, x, Wo, bo, ln1_g, ln1_b, W1, b1, W2, b2, ln2_g, ln2_b):
    grid = (N // ROW_BLK,)
    full = lambda r, c: pl.BlockSpec((r, c), lambda i: (0, 0))
    return pl.pallas_call(
        _post_body,
        grid=grid,
        in_specs=[
            pl.BlockSpec((ROW_BLK, NDIM), lambda i: (i, 0)),
            pl.BlockSpec((ROW_BLK, H), lambda i: (i, 0)),
            pl.BlockSpec((ROW_BLK, EDIM), lambda i: (i, 0)),
            full(NDIM, EDIM), full(1, EDIM), full(1, EDIM), full(1, EDIM),
            full(EDIM, 4 * EDIM), full(1, 4 * EDIM),
            full(4 * EDIM, EDIM), full(1, EDIM),
            full(1, EDIM), full(1, EDIM),
        ],
        out_specs=pl.BlockSpec((ROW_BLK, EDIM), lambda i: (i, 0)),
        out_shape=jax.ShapeDtypeStruct((N, EDIM), jnp.float32),
    )(wv, z, x, Wo, bo.reshape(1, EDIM), ln1_g.reshape(1, EDIM),
      ln1_b.reshape(1, EDIM), W1, b1.reshape(1, 4 * EDIM), W2,
      b2.reshape(1, EDIM), ln2_g.reshape(1, EDIM), ln2_b.reshape(1, EDIM))


def kernel(x, src_x, dst_x, edge_index, Wq, bq, Wk, Wv, Wo, bo, ln1_g, ln1_b,
           W1, b1, W2, b2, ln2_g, ln2_b):
    q3, k3, v3 = _qkv(x, src_x, dst_x, Wq, bq, Wk, Wv)
    qf = q3.reshape(2 * N, HALF)
    kf = k3.reshape(2 * N, HALF)
    vf = v3.reshape(2 * N, HALF)
    src = edge_index[0]
    dst = edge_index[1]
    off = jnp.array([[0], [N]], jnp.int32)
    srcp = (src[None, :] + off).reshape(2, E // CH, CH)
    dstp = (dst[None, :] + off).reshape(2, E // CH, CH)
    dstr = jnp.broadcast_to(dst.reshape(1, E // CH, CH), (2, E // CH, CH))
    comb = jnp.stack([srcp, dstp, dstr], axis=2).reshape(2 * E * 3)
    accum = _edge_sc(kf, qf, vf, comb)
    a0 = accum[:ACC_ROWS]
    a1 = accum[ACC_ROWS:]
    wv = jnp.concatenate([a0[:N], a1[:N]], axis=1)
    z0 = a0[N_PAD:].reshape(ZROWS * 32, 4)[:N]
    z1 = a1[N_PAD:].reshape(ZROWS * 32, 4)[:N]
    z = jnp.concatenate([z0, z1], axis=1)
    return _post(wv, z, x, Wo, bo, ln1_g, ln1_b, W1, b1, W2, b2,
                 ln2_g, ln2_b)


# speedup vs baseline: 19.6110x; 1.0239x over previous
import dataclasses
import functools
import math
import jax
import jax.numpy as jnp
from jax import lax
from jax.experimental import pallas as pl
from jax.experimental.pallas import tpu as pltpu
from jax.experimental.pallas import tpu_sc as plsc

N = 10000
E = 320000
EDIM = 128
NDIM = 256
H = 8
DK = NDIM // H

ROW_BLK = 1000
HALF = 128
CH = 32
GRP = 16
NTILES = 16
EPT = E // NTILES
N_PAD = 10112
ZROWS = 512
ACC_ROWS = N_PAD + ZROWS
ROWS_PT = ACC_ROWS // NTILES
_INV_SQRT_DK = 1.0 / math.sqrt(DK)


def _qkv_body(x_ref, src_ref, dst_ref, wq_ref, bq_ref, wk_ref, wv_ref,
              q_ref, k_ref, v_ref):
    x = x_ref[...]
    q = jnp.dot(x, wq_ref[...], preferred_element_type=jnp.float32) \
        + bq_ref[...] + src_ref[...]
    k = jnp.dot(x, wk_ref[...],
                preferred_element_type=jnp.float32) * _INV_SQRT_DK
    v = jnp.dot(x, wv_ref[...], preferred_element_type=jnp.float32) \
        + dst_ref[...]
    q_ref[0], q_ref[1] = q[:, :HALF], q[:, HALF:]
    k_ref[0], k_ref[1] = k[:, :HALF], k[:, HALF:]
    v_ref[0], v_ref[1] = v[:, :HALF], v[:, HALF:]


def _qkv(x, src_x, dst_x, Wq, bq, Wk, Wv):
    grid = (N // ROW_BLK,)
    row_spec = pl.BlockSpec((ROW_BLK, EDIM), lambda i: (i, 0))
    row_spec_n = pl.BlockSpec((ROW_BLK, NDIM), lambda i: (i, 0))
    w_spec = pl.BlockSpec((EDIM, NDIM), lambda i: (0, 0))
    b_spec = pl.BlockSpec((1, NDIM), lambda i: (0, 0))
    out_spec = pl.BlockSpec((2, ROW_BLK, HALF), lambda i: (0, i, 0))
    out_sds = jax.ShapeDtypeStruct((2, N, HALF), jnp.float32)
    return pl.pallas_call(
        _qkv_body,
        grid=grid,
        in_specs=[row_spec, row_spec_n, row_spec_n, w_spec, b_spec, w_spec,
                  w_spec],
        out_specs=[out_spec, out_spec, out_spec],
        out_shape=[out_sds, out_sds, out_sds],
    )(x, src_x, dst_x, Wq, bq.reshape(1, NDIM), Wk, Wv)


_GDN = lax.GatherDimensionNumbers(offset_dims=(), collapsed_slice_dims=(0,),
                                  start_index_map=(0,))


def _lane_shuffle(x, idx):
    return lax.gather(x, idx[:, None], _GDN, (1,),
                      mode=lax.GatherScatterMode.PROMISE_IN_BOUNDS)


def _edge_sc_body(kf_hbm, qf_hbm, vf_hbm, comb_hbm, out_hbm,
                  idx0, idx1, sidx0, sidx1, szc0, szc1, dc0, dc1,
                  k0, k1, q0, q1, v0, v1, pay0, pay1,
                  acc,
                  si0, si1, sk0, sk1, ss0, ss1):
    idx_b, sidx_b, szc_v = (idx0, idx1), (sidx0, sidx1), (szc0, szc1)
    dc_v = (dc0, dc1)
    k_b, q_b, v_b, pay_b = (k0, k1), (q0, q1), (v0, v1), (pay0, pay1)
    sem_idx, sem_kqv, sem_sc = (si0, si1), (sk0, sk1), (ss0, ss1)
    cid = lax.axis_index("c")
    sid = lax.axis_index("s")
    zeros16 = jnp.zeros((GRP,), jnp.float32)
    iota16 = lax.iota(jnp.int32, GRP)
    nchunks = EPT // CH

    @pl.loop(0, 2 * CH)
    def _zrow(r):
        for j in range(HALF // GRP):
            pay_b[0][r, pl.ds(j * GRP, GRP)] = zeros16
            pay_b[1][r, pl.ds(j * GRP, GRP)] = zeros16

    row0 = sid * ROWS_PT
    for j in range(ROWS_PT // (2 * CH)):
        pltpu.sync_copy(pay_b[0], acc.at[pl.ds(row0 + j * 2 * CH, 2 * CH)])
    _rem = ROWS_PT % (2 * CH)
    if _rem:
        pltpu.sync_copy(pay_b[0].at[pl.ds(0, _rem)],
                        acc.at[pl.ds(row0 + ROWS_PT - _rem, _rem)])

    plsc.subcore_barrier()

    def idx_descr(ec, p):
        off = (cid * NTILES * nchunks + sid * nchunks + ec // CH) * 3 * CH
        return pltpu.make_async_copy(comb_hbm.at[pl.ds(off, 3 * CH)],
                                     idx_b[p], sem_idx[p])

    def gather_descr(p):
        src_ref = idx_b[p].at[pl.ds(0, CH)]
        dq_ref = idx_b[p].at[pl.ds(CH, CH)]
        return (pltpu.make_async_copy(kf_hbm.at[src_ref], k_b[p],
                                      sem_kqv[p]),
                pltpu.make_async_copy(qf_hbm.at[dq_ref], q_b[p],
                                      sem_kqv[p]),
                pltpu.make_async_copy(vf_hbm.at[src_ref], v_b[p],
                                      sem_kqv[p]))

    def sc_start(p):
        pltpu.async_copy(pay_b[p], acc.at[sidx_b[p]], sem_sc[p], add=True)

    def sc_wait(p):
        pltpu.make_async_copy(pay_b[p], acc.at[sidx_b[p]], sem_sc[p]).wait()

    zmask = iota16 < 4
    perms = [iota16 ^ kk for kk in (1, 2, 4, 8)]

    def compute_group(p, g):
        dvec = dc_v[p][pl.ds(g, GRP)]
        sidx_b[p][pl.ds(g, GRP)] = dvec
        sidx_b[p][pl.ds(CH + g, GRP)] = \
            N_PAD + lax.shift_right_logical(dvec, 5)
        zcol0 = jnp.bitwise_and(dvec, 31) * 4
        szc_v[p][pl.ds(g, GRP)] = zcol0
        for i in range(GRP):
            e = g + i
            prods = [k_b[p][e, pl.ds(16 * j, 16)]
                     * q_b[p][e, pl.ds(16 * j, 16)]
                     for j in range(HALF // GRP)]
            svs = []
            for h in range(H // 2):
                t = prods[2 * h] + prods[2 * h + 1]
                for pm in perms:
                    t = t + _lane_shuffle(t, pm)
                sv = jnp.exp(jnp.clip(t, -10.0, 10.0))
                svs.append(sv)
                for j in (2 * h, 2 * h + 1):
                    pay_b[p][e, pl.ds(16 * j, 16)] = \
                        v_b[p][e, pl.ds(16 * j, 16)] * sv
            sv_all = jnp.where(iota16 == 0, svs[0],
                               jnp.where(iota16 == 1, svs[1],
                                         jnp.where(iota16 == 2, svs[2],
                                                   svs[3])))
            rows = jnp.broadcast_to(CH + e, (GRP,))
            cols = jnp.broadcast_to(zcol0[i], (GRP,)) + iota16
            plsc.store_scatter(pay_b[p], [rows, cols], sv_all, mask=zmask)

    def clear_zpay(p):
        for g in range(0, CH, GRP):
            eids = CH + g + iota16
            zcol0 = szc_v[p][pl.ds(g, GRP)]
            for h in range(H // 2):
                plsc.store_scatter(pay_b[p], [eids, zcol0 + h], zeros16)

    def body(ec, p, first=False, static_guards=False):
        q = 1 - p
        for d in gather_descr(p):
            d.wait()
        for g in range(0, CH, GRP):
            dc_v[p][pl.ds(g, GRP)] = idx_b[p][pl.ds(2 * CH + g, GRP)]
        def prefetch_idx2():
            idx_descr(ec + 2 * CH, p).start()
        if static_guards:
            if ec + 2 * CH < EPT:
                prefetch_idx2()
        else:
            pl.when(ec + 2 * CH < EPT)(prefetch_idx2)
        def issue_next():
            idx_descr(ec + CH, q).wait()
            for d in gather_descr(q):
                d.start()
        if static_guards:
            if ec + CH < EPT:
                issue_next()
        else:
            pl.when(ec + CH < EPT)(issue_next)
        compute_group(p, 0)
        if not first:
            sc_wait(q)
            clear_zpay(q)
        compute_group(p, GRP)
        sc_start(p)

    idx_descr(0, 0).start()
    idx_descr(0, 0).wait()
    for d in gather_descr(0):
        d.start()
    idx_descr(CH, 1).start()
    body(0, 0, first=True, static_guards=True)

    @pl.loop(CH, EPT, step=2 * CH)
    def _pair(e0):
        body(e0, 1)
        body(e0 + CH, 0)

    sc_wait(0)

    plsc.subcore_barrier()

    for j in range(ROWS_PT // (2 * CH)):
        r = row0 + j * 2 * CH
        pltpu.sync_copy(acc.at[pl.ds(r, 2 * CH)], pay_b[0])
        pltpu.sync_copy(pay_b[0],
                        out_hbm.at[pl.ds(cid * ACC_ROWS + r, 2 * CH)])
    if _rem:
        r = row0 + ROWS_PT - _rem
        pltpu.sync_copy(acc.at[pl.ds(r, _rem)], pay_b[0].at[pl.ds(0, _rem)])
        pltpu.sync_copy(pay_b[0].at[pl.ds(0, _rem)],
                        out_hbm.at[pl.ds(cid * ACC_ROWS + r, _rem)])


def _edge_sc(kf, qf, vf, comb):
    mesh = plsc.VectorSubcoreMesh(core_axis_name="c", subcore_axis_name="s")
    cp = pltpu.CompilerParams()
    if "needs_layout_passes" in pltpu.CompilerParams.__dataclass_fields__:
        cp = dataclasses.replace(cp, needs_layout_passes=False)
    kern = functools.partial(
        pl.kernel,
        mesh=mesh,
        compiler_params=cp,
        out_type=jax.ShapeDtypeStruct((2 * ACC_ROWS, HALF), jnp.float32),
        scratch_types=(
            [pltpu.VMEM((3 * CH,), jnp.int32)] * 2
            + [pltpu.VMEM((2 * CH,), jnp.int32)] * 2
            + [pltpu.VMEM((CH,), jnp.int32)] * 4
            + [pltpu.VMEM((CH, HALF), jnp.float32)] * 6
            + [pltpu.VMEM((2 * CH, HALF), jnp.float32)] * 2
            + [pltpu.VMEM_SHARED((ACC_ROWS, HALF), jnp.float32)]
            + [pltpu.SemaphoreType.DMA] * 6
        ),
    )(_edge_sc_body)
    return kern(kf, qf, vf, comb)


def _ln(x, g, b, eps=1e-5):
    m = jnp.mean(x, axis=-1, keepdims=True)
    v = jnp.mean((x - m) ** 2, axis=-1, keepdims=True)
    return (x - m) / jnp.sqrt(v + eps) * g + b


def _post_body(wv_ref, z_ref, x_ref, wo_ref, bo_ref, ln1g_ref, ln1b_ref,
               w1_ref, b1_ref, w2_ref, b2_ref, ln2g_ref, ln2b_ref, out_ref):
    wv = wv_ref[...]
    z = z_ref[...]
    zb = jnp.repeat(z, DK, axis=1)
    o = wv / (zb + 1e-12)
    x = x_ref[...]
    h1 = x + jnp.dot(o, wo_ref[...], preferred_element_type=jnp.float32) \
        + bo_ref[...]
    h1 = _ln(h1, ln1g_ref[...], ln1b_ref[...])
    h2 = jnp.maximum(
        jnp.dot(h1, w1_ref[...], preferred_element_type=jnp.float32)
        + b1_ref[...], 0.0)
    h3 = h1 + jnp.dot(h2, w2_ref[...], preferred_element_type=jnp.float32) \
        + b2_ref[...]
    out_ref[...] = _ln(h3, ln2g_ref[...], ln2b_ref[...])


def _post(wv, z, x, Wo, bo, ln1_g, ln1_b, W1, b1, W2, b2, ln2_g, ln2_b):
    grid = (N // ROW_BLK,)
    full = lambda r, c: pl.BlockSpec((r, c), lambda i: (0, 0))
    return pl.pallas_call(
        _post_body,
        grid=grid,
        in_specs=[
            pl.BlockSpec((ROW_BLK, NDIM), lambda i: (i, 0)),
            pl.BlockSpec((ROW_BLK, H), lambda i: (i, 0)),
            pl.BlockSpec((ROW_BLK, EDIM), lambda i: (i, 0)),
            full(NDIM, EDIM), full(1, EDIM), full(1, EDIM), full(1, EDIM),
            full(EDIM, 4 * EDIM), full(1, 4 * EDIM),
            full(4 * EDIM, EDIM), full(1, EDIM),
            full(1, EDIM), full(1, EDIM),
        ],
        out_specs=pl.BlockSpec((ROW_BLK, EDIM), lambda i: (i, 0)),
        out_shape=jax.ShapeDtypeStruct((N, EDIM), jnp.float32),
    )(wv, z, x, Wo, bo.reshape(1, EDIM), ln1_g.reshape(1, EDIM),
      ln1_b.reshape(1, EDIM), W1, b1.reshape(1, 4 * EDIM), W2,
      b2.reshape(1, EDIM), ln2_g.reshape(1, EDIM), ln2_b.reshape(1, EDIM))


def kernel(x, src_x, dst_x, edge_index, Wq, bq, Wk, Wv, Wo, bo, ln1_g, ln1_b,
           W1, b1, W2, b2, ln2_g, ln2_b):
    q3, k3, v3 = _qkv(x, src_x, dst_x, Wq, bq, Wk, Wv)
    qf = q3.reshape(2 * N, HALF)
    kf = k3.reshape(2 * N, HALF)
    vf = v3.reshape(2 * N, HALF)
    src = edge_index[0]
    dst = edge_index[1]
    off = jnp.array([[0], [N]], jnp.int32)
    srcp = (src[None, :] + off).reshape(2, E // CH, CH)
    dstp = (dst[None, :] + off).reshape(2, E // CH, CH)
    dstr = jnp.broadcast_to(dst.reshape(1, E // CH, CH), (2, E // CH, CH))
    comb = jnp.stack([srcp, dstp, dstr], axis=2).reshape(2 * E * 3)
    accum = _edge_sc(kf, qf, vf, comb)
    a0 = accum[:ACC_ROWS]
    a1 = accum[ACC_ROWS:]
    wv = jnp.concatenate([a0[:N], a1[:N]], axis=1)
    z0 = a0[N_PAD:].reshape(ZROWS * 32, 4)[:N]
    z1 = a1[N_PAD:].reshape(ZROWS * 32, 4)[:N]
    z = jnp.concatenate([z0, z1], axis=1)
    return _post(wv, z, x, Wo, bo, ln1_g, ln1_b, W1, b1, W2, b2,
                 ln2_g, ln2_b)
